# Initial kernel scaffold; baseline (speedup 1.0000x reference)
import jax
import jax.numpy as jnp
from jax.experimental import pallas as pl


def kernel(x, w_up, b_up, gamma1, beta1, alpha1, w_res, b_res, gamma2, beta2, alpha2):
    raise NotImplementedError("write your pallas kernel here")


# speedup vs baseline: 7.0512x; 7.0512x over previous
import functools

import jax
import jax.numpy as jnp
import numpy as np
from jax import lax
from jax.experimental import pallas as pl
from jax.experimental.pallas import tpu as pltpu

_EPS = 1e-5


def _up_kernel(x0_ref, x1_ref, w_ref, y_ref, s_ref, *, D, H, W):
    d = pl.program_id(0) % D
    x0 = x0_ref[0]
    x1 = jnp.where(d < D - 1, x1_ref[0], jnp.zeros_like(x1_ref[0]))
    lane = lax.broadcasted_iota(jnp.int32, (1, H * W), 1)
    h = lane // W
    w = lane % W
    rows = []
    for xs in (x0, x1):
        for sh in (0, 1):
            for sw in (0, 1):
                off = sh * W + sw
                if off == 0:
                    t = xs
                else:
                    t = jnp.concatenate(
                        [xs[:, off:], jnp.zeros((xs.shape[0], off), xs.dtype)],
                        axis=1)
                valid = (h + sh < H) & (w + sw < W)
                rows.append(jnp.where(valid, t, jnp.zeros_like(t)))
    xm = jnp.concatenate(rows, axis=0)
    y = jnp.dot(w_ref[...], xm, preferred_element_type=jnp.float32)
    y_ref[...] = y
    sums = jnp.sum(y, axis=1, keepdims=True)
    sqs = jnp.sum(y * y, axis=1, keepdims=True)
    s_ref[...] = jnp.concatenate([sums, sqs], axis=1)[None]


def _phase_weight(w_up):
    Cout = w_up.shape[1]
    Cin = w_up.shape[0]
    sel = np.zeros((2, 2, 3), np.float32)
    sel[0, 0, 1] = 1.0
    sel[1, 0, 2] = 1.0
    sel[1, 1, 0] = 1.0
    sel = jnp.asarray(sel)
    w8 = jnp.einsum('PSa,QTb,RUc,ioabc->PQRoSTUi', sel, sel, sel,
                    w_up.astype(jnp.float32))
    return w8.reshape(8 * Cout, 8 * Cin)


def _res_kernel(ym_ref, y0_ref, yp_ref, sc_ref, sh_ref, al_ref, w_ref,
                z_ref, s_ref, *, Do, Ho, Wo):
    do = pl.program_id(0) % Do
    sc = sc_ref[...]
    sh = sh_ref[...]
    al = al_ref[0, 0]

    def act(ref, valid):
        t = ref[0] * sc + sh
        t = jnp.where(t > 0, t, al * t)
        t = jnp.where(valid, t, jnp.zeros_like(t))
        return t.astype(jnp.bfloat16)

    slabs = (act(ym_ref, do > 0), act(y0_ref, do >= 0), act(yp_ref, do < Do - 1))
    lane = lax.broadcasted_iota(jnp.int32, (1, Ho * Wo), 1)
    h = lane // Wo
    w = lane % Wo
    rows = []
    for s in slabs:
        for kh in (-1, 0, 1):
            for kw in (-1, 0, 1):
                off = kh * Wo + kw
                if off > 0:
                    t = jnp.concatenate(
                        [s[:, off:], jnp.zeros((s.shape[0], off), s.dtype)],
                        axis=1)
                elif off < 0:
                    t = jnp.concatenate(
                        [jnp.zeros((s.shape[0], -off), s.dtype), s[:, :off]],
                        axis=1)
                else:
                    t = s
                valid = ((h + kh >= 0) & (h + kh < Ho)
                         & (w + kw >= 0) & (w + kw < Wo))
                rows.append(jnp.where(valid, t, jnp.zeros_like(t)))
    xm = jnp.concatenate(rows, axis=0)
    z = jnp.dot(w_ref[...], xm, preferred_element_type=jnp.float32)
    z_ref[0] = z
    sums = jnp.sum(z, axis=1, keepdims=True)
    sqs = jnp.sum(z * z, axis=1, keepdims=True)
    s_ref[...] = jnp.concatenate([sums, sqs], axis=1)[None]


def _final_kernel(z_ref, y_ref, sc1_ref, sh1_ref, al1_ref,
                  sc2_ref, sh2_ref, al2_ref, o_ref):
    t1 = y_ref[0, 0] * sc1_ref[...] + sh1_ref[...]
    a1 = al1_ref[0, 0, 0]
    r = jnp.where(t1 > 0, t1, a1 * t1)
    t2 = z_ref[0, 0] * sc2_ref[...] + sh2_ref[...]
    a2 = al2_ref[0, 0, 0]
    o_ref[0, :, 0] = jnp.where(t2 > 0, t2, a2 * t2) + r


def _finalize_bn(s, count, gamma, beta):
    mean = s[:, 0] / count
    var = jnp.maximum(s[:, 1] / count - mean * mean, 0.0)
    scale = gamma.astype(jnp.float32) * lax.rsqrt(var + _EPS)
    shift = beta.astype(jnp.float32) - mean * scale
    return scale.reshape(-1, 1), shift.reshape(-1, 1)


def kernel(x, w_up, b_up, gamma1, beta1, alpha1,
           w_res, b_res, gamma2, beta2, alpha2):
    N, Cin, D, H, W = x.shape
    Cout = w_up.shape[1]
    Do, Ho, Wo = 2 * D, 2 * H, 2 * W
    HW, HWo = H * W, Ho * Wo
    count = N * Do * HWo

    x_t = jnp.transpose(x, (0, 2, 1, 3, 4)).reshape(N * D, Cin, HW)
    x_t = x_t.astype(jnp.bfloat16)
    w8 = _phase_weight(w_up).astype(jnp.bfloat16)
    ga = N * D
    slab = pl.BlockSpec((1, Cin, HW), lambda i: (i, 0, 0))
    slab_p = pl.BlockSpec((1, Cin, HW), lambda i: (jnp.minimum(i + 1, ga - 1), 0, 0))
    y_ph, st1 = pl.pallas_call(
        functools.partial(_up_kernel, D=D, H=H, W=W),
        out_shape=(jax.ShapeDtypeStruct((8 * Cout, ga * HW), jnp.float32),
                   jax.ShapeDtypeStruct((ga, 8 * Cout, 2), jnp.float32)),
        grid=(ga,),
        in_specs=[slab, slab_p,
                  pl.BlockSpec((8 * Cout, 8 * Cin), lambda i: (0, 0))],
        out_specs=(pl.BlockSpec((8 * Cout, HW), lambda i: (0, i)),
                   pl.BlockSpec((1, 8 * Cout, 2), lambda i: (i, 0, 0))),
        compiler_params=pltpu.CompilerParams(dimension_semantics=("parallel",)),
    )(x_t, x_t, w8)

    s1 = st1.sum(axis=0).reshape(8, Cout, 2).sum(axis=0)
    sc1, sh1 = _finalize_bn(s1, count, gamma1, beta1)
    al1 = jnp.full((1, 1), alpha1, jnp.float32)

    y_r = (y_ph.reshape(2, 2, 2, Cout, N, D, H, W)
           .transpose(4, 5, 0, 3, 6, 1, 7, 2)
           .reshape(N * Do, Cout, HWo))

    w_r = jnp.transpose(w_res, (0, 2, 3, 4, 1)).reshape(Cout, 27 * Cout)
    w_r = w_r.astype(jnp.bfloat16)
    gb = N * Do
    yslab = pl.BlockSpec((1, Cout, HWo), lambda i: (i, 0, 0))
    yslab_m = pl.BlockSpec((1, Cout, HWo), lambda i: (jnp.maximum(i - 1, 0), 0, 0))
    yslab_p = pl.BlockSpec((1, Cout, HWo), lambda i: (jnp.minimum(i + 1, gb - 1), 0, 0))
    cvec = pl.BlockSpec((Cout, 1), lambda i: (0, 0))
    one = pl.BlockSpec((1, 1), lambda i: (0, 0))
    z, st2 = pl.pallas_call(
        functools.partial(_res_kernel, Do=Do, Ho=Ho, Wo=Wo),
        out_shape=(jax.ShapeDtypeStruct((gb, Cout, HWo), jnp.float32),
                   jax.ShapeDtypeStruct((gb, Cout, 2), jnp.float32)),
        grid=(gb,),
        in_specs=[yslab_m, yslab, yslab_p, cvec, cvec, one,
                  pl.BlockSpec((Cout, 27 * Cout), lambda i: (0, 0))],
        out_specs=(yslab, pl.BlockSpec((1, Cout, 2), lambda i: (i, 0, 0))),
        compiler_params=pltpu.CompilerParams(dimension_semantics=("parallel",)),
    )(y_r, y_r, y_r, sc1, sh1, al1, w_r)

    sc2, sh2 = _finalize_bn(st2.sum(axis=0), count, gamma2, beta2)
    al2 = jnp.full((1, 1), alpha2, jnp.float32)

    lw = HWo // 8
    z5 = z.reshape(N, Do, Cout, 8, lw)
    y5 = y_r.reshape(N, Do, Cout, 8, lw)
    slab5 = pl.BlockSpec((1, 1, Cout, 8, lw), lambda n, d: (n, d, 0, 0, 0))
    cvec2 = pl.BlockSpec((Cout, 1, 1), lambda n, d: (0, 0, 0))
    one2 = pl.BlockSpec((1, 1, 1), lambda n, d: (0, 0, 0))
    sc1c, sh1c = sc1.reshape(Cout, 1, 1), sh1.reshape(Cout, 1, 1)
    sc2c, sh2c = sc2.reshape(Cout, 1, 1), sh2.reshape(Cout, 1, 1)
    al1c = al1.reshape(1, 1, 1)
    al2c = al2.reshape(1, 1, 1)
    out = pl.pallas_call(
        _final_kernel,
        out_shape=jax.ShapeDtypeStruct((N, Cout, Do, 8, lw), jnp.float32),
        grid=(N, Do),
        in_specs=[slab5, slab5, cvec2, cvec2, one2, cvec2, cvec2, one2],
        out_specs=pl.BlockSpec((1, Cout, 1, 8, lw), lambda n, d: (n, 0, d, 0, 0)),
        compiler_params=pltpu.CompilerParams(
            dimension_semantics=("parallel", "parallel")),
    )(z5, y5, sc1c, sh1c, al1c, sc2c, sh2c, al2c)
    return out.reshape(N, Cout, Do, Ho, Wo)


# speedup vs baseline: 7.8256x; 1.1098x over previous
import functools

import jax
import jax.numpy as jnp
import numpy as np
from jax import lax
from jax.experimental import pallas as pl
from jax.experimental.pallas import tpu as pltpu

_EPS = 1e-5


def _up_kernel(x0_ref, x1_ref, w_ref, y_ref, s_ref, *, D, H, W):
    d = pl.program_id(0) % D
    x0 = x0_ref[0]
    x1 = jnp.where(d < D - 1, x1_ref[0], jnp.zeros_like(x1_ref[0]))
    lane = lax.broadcasted_iota(jnp.int32, (1, H * W), 1)
    h = lane // W
    w = lane % W
    rows = []
    for xs in (x0, x1):
        for sh in (0, 1):
            for sw in (0, 1):
                off = sh * W + sw
                if off == 0:
                    t = xs
                else:
                    t = jnp.concatenate(
                        [xs[:, off:], jnp.zeros((xs.shape[0], off), xs.dtype)],
                        axis=1)
                valid = (h + sh < H) & (w + sw < W)
                rows.append(jnp.where(valid, t, jnp.zeros_like(t)))
    xm = jnp.concatenate(rows, axis=0)
    y = jnp.dot(w_ref[...], xm, preferred_element_type=jnp.float32)
    y_ref[0] = y.astype(jnp.bfloat16)
    sums = jnp.sum(y, axis=1, keepdims=True)
    sqs = jnp.sum(y * y, axis=1, keepdims=True)
    s_ref[...] = jnp.concatenate([sums, sqs], axis=1)[None]


def _phase_weight(w_up):
    sel = np.zeros((2, 2, 3), np.float32)
    sel[0, 0, 1] = 1.0
    sel[1, 0, 2] = 1.0
    sel[1, 1, 0] = 1.0
    sel = jnp.asarray(sel)
    w8 = jnp.einsum('PSa,QTb,RUc,ioabc->PQRoSTUi', sel, sel, sel,
                    w_up.astype(jnp.float32))
    Cout, Cin = w_up.shape[1], w_up.shape[0]
    return w8.reshape(8 * Cout, 8 * Cin)


def _res_kernel(hm_ref, c_ref, hp_ref, w_ref, z_ref, s_ref, *, Do, Ho, Wo):
    i = pl.program_id(0)
    do0 = (2 * i) % Do
    s0 = jnp.where(do0 > 0, hm_ref[0], jnp.zeros_like(hm_ref[0]))
    s1 = c_ref[0]
    s2 = c_ref[1]
    s3 = jnp.where(do0 < Do - 2, hp_ref[0], jnp.zeros_like(hp_ref[0]))
    slabs = (s0, s1, s2, s3)

    lane = lax.broadcasted_iota(jnp.int32, (1, Ho * Wo), 1)
    h = lane // Wo
    w = lane % Wo
    rows = [None] * 27
    for kh in (0, 1, 2):
        for kw in (0, 1, 2):
            off = (kh - 1) * Wo + (kw - 1)
            valid = ((h + kh - 1 >= 0) & (h + kh - 1 < Ho)
                     & (w + kw - 1 >= 0) & (w + kw - 1 < Wo))
            shifted = []
            for s in slabs:
                if off > 0:
                    t = jnp.concatenate(
                        [s[:, off:], jnp.zeros((s.shape[0], off), s.dtype)],
                        axis=1)
                elif off < 0:
                    t = jnp.concatenate(
                        [jnp.zeros((s.shape[0], -off), s.dtype), s[:, :off]],
                        axis=1)
                else:
                    t = s
                shifted.append(jnp.where(valid, t, jnp.zeros_like(t)))
            for kd in (0, 1, 2):
                rows[(kd * 3 + kh) * 3 + kw] = jnp.concatenate(
                    [shifted[kd], shifted[kd + 1]], axis=1)
    xm = jnp.concatenate(rows, axis=0)
    z = jnp.dot(w_ref[...], xm, preferred_element_type=jnp.float32)
    z_ref[0] = z[:, :Ho * Wo]
    z_ref[1] = z[:, Ho * Wo:]
    sums = jnp.sum(z, axis=1, keepdims=True)
    sqs = jnp.sum(z * z, axis=1, keepdims=True)
    s_ref[...] = jnp.concatenate([sums, sqs], axis=1)[None]


def _final_kernel(z_ref, y_ref, sc2_ref, sh2_ref, al2_ref, o_ref):
    a2 = al2_ref[0, 0, 0]
    for k in range(z_ref.shape[1]):
        t2 = z_ref[0, k] * sc2_ref[...] + sh2_ref[...]
        o_ref[0, :, k] = (jnp.where(t2 > 0, t2, a2 * t2)
                          + y_ref[0, k].astype(jnp.float32))


def _finalize_bn(s, count, gamma, beta):
    mean = s[:, 0] / count
    var = jnp.maximum(s[:, 1] / count - mean * mean, 0.0)
    scale = gamma.astype(jnp.float32) * lax.rsqrt(var + _EPS)
    shift = beta.astype(jnp.float32) - mean * scale
    return scale.reshape(-1, 1), shift.reshape(-1, 1)


def kernel(x, w_up, b_up, gamma1, beta1, alpha1,
           w_res, b_res, gamma2, beta2, alpha2):
    N, Cin, D, H, W = x.shape
    Cout = w_up.shape[1]
    Do, Ho, Wo = 2 * D, 2 * H, 2 * W
    HW, HWo = H * W, Ho * Wo
    count = N * Do * HWo

    x_t = (x.astype(jnp.bfloat16)
           .transpose(0, 2, 1, 3, 4).reshape(N * D, Cin, HW))
    w8 = _phase_weight(w_up).astype(jnp.bfloat16)
    ga = N * D
    slab = pl.BlockSpec((1, Cin, HW), lambda i: (i, 0, 0))
    slab_p = pl.BlockSpec((1, Cin, HW), lambda i: (jnp.minimum(i + 1, ga - 1), 0, 0))
    y_ph, st1 = pl.pallas_call(
        functools.partial(_up_kernel, D=D, H=H, W=W),
        out_shape=(jax.ShapeDtypeStruct((ga, 8 * Cout, HW), jnp.bfloat16),
                   jax.ShapeDtypeStruct((ga, 8 * Cout, 2), jnp.float32)),
        grid=(ga,),
        in_specs=[slab, slab_p,
                  pl.BlockSpec((8 * Cout, 8 * Cin), lambda i: (0, 0))],
        out_specs=(pl.BlockSpec((1, 8 * Cout, HW), lambda i: (i, 0, 0)),
                   pl.BlockSpec((1, 8 * Cout, 2), lambda i: (i, 0, 0))),
        compiler_params=pltpu.CompilerParams(dimension_semantics=("parallel",)),
    )(x_t, x_t, w8)

    s1 = st1.sum(axis=0).reshape(8, Cout, 2).sum(axis=0)
    sc1, sh1 = _finalize_bn(s1, count, gamma1, beta1)

    y_d = (y_ph.reshape(N, D, 2, 2, 2, Cout, H, W)
           .transpose(0, 1, 2, 5, 6, 3, 7, 4)
           .reshape(N * Do, Cout, HWo))
    t1 = y_d.astype(jnp.float32) * sc1[None] + sh1[None]
    y_act = jnp.where(t1 > 0, t1, alpha1 * t1).astype(jnp.bfloat16)

    w_r = jnp.transpose(w_res, (0, 2, 3, 4, 1)).reshape(Cout, 27 * Cout)
    w_r = w_r.astype(jnp.bfloat16)
    nrow = N * Do
    gb = nrow // 2
    center = pl.BlockSpec((2, Cout, HWo), lambda i: (i, 0, 0))
    halo_m = pl.BlockSpec((1, Cout, HWo),
                          lambda i: (jnp.maximum(2 * i - 1, 0), 0, 0))
    halo_p = pl.BlockSpec((1, Cout, HWo),
                          lambda i: (jnp.minimum(2 * i + 2, nrow - 1), 0, 0))
    z, st2 = pl.pallas_call(
        functools.partial(_res_kernel, Do=Do, Ho=Ho, Wo=Wo),
        out_shape=(jax.ShapeDtypeStruct((nrow, Cout, HWo), jnp.float32),
                   jax.ShapeDtypeStruct((gb, Cout, 2), jnp.float32)),
        grid=(gb,),
        in_specs=[halo_m, center, halo_p,
                  pl.BlockSpec((Cout, 27 * Cout), lambda i: (0, 0))],
        out_specs=(center, pl.BlockSpec((1, Cout, 2), lambda i: (i, 0, 0))),
        compiler_params=pltpu.CompilerParams(dimension_semantics=("parallel",)),
    )(y_act, y_act, y_act, w_r)

    sc2, sh2 = _finalize_bn(st2.sum(axis=0), count, gamma2, beta2)

    lw = HWo // 8
    z5 = z.reshape(N, Do, Cout, 8, lw)
    y5 = y_act.reshape(N, Do, Cout, 8, lw)
    slab5 = pl.BlockSpec((1, 2, Cout, 8, lw), lambda n, d: (n, d, 0, 0, 0))
    cvec = pl.BlockSpec((Cout, 1, 1), lambda n, d: (0, 0, 0))
    one = pl.BlockSpec((1, 1, 1), lambda n, d: (0, 0, 0))
    out = pl.pallas_call(
        _final_kernel,
        out_shape=jax.ShapeDtypeStruct((N, Cout, Do, 8, lw), jnp.float32),
        grid=(N, Do // 2),
        in_specs=[slab5, slab5, cvec, cvec, one],
        out_specs=pl.BlockSpec((1, Cout, 2, 8, lw),
                               lambda n, d: (n, 0, d, 0, 0)),
        compiler_params=pltpu.CompilerParams(
            dimension_semantics=("parallel", "parallel")),
    )(z5, y5, sc2.reshape(Cout, 1, 1), sh2.reshape(Cout, 1, 1),
      jnp.full((1, 1, 1), alpha2, jnp.float32))
    return out.reshape(N, Cout, Do, Ho, Wo)


# speedup vs baseline: 10.8601x; 1.3878x over previous
import functools

import jax
import jax.numpy as jnp
import numpy as np
from jax import lax
from jax.experimental import pallas as pl
from jax.experimental.pallas import tpu as pltpu

_EPS = 1e-5


def _shift_lanes(s, off):
    if off == 0:
        return s
    if off > 0:
        return jnp.concatenate(
            [s[:, off:], jnp.zeros((s.shape[0], off), s.dtype)], axis=1)
    return jnp.concatenate(
        [jnp.zeros((s.shape[0], -off), s.dtype), s[:, :off]], axis=1)


def _up_kernel(c_ref, hp_ref, w_ref, y_ref, s_ref, *, D, H, W):
    d0 = (2 * pl.program_id(0)) % D
    s0 = c_ref[0]
    s1 = c_ref[1]
    s2 = jnp.where(d0 < D - 2, hp_ref[0], jnp.zeros_like(hp_ref[0]))
    slabs = (s0, s1, s2)
    lane = lax.broadcasted_iota(jnp.int32, (1, H * W), 1)
    h = lane // W
    w = lane % W
    rows = []
    for sd in (0, 1):
        for sh in (0, 1):
            for sw in (0, 1):
                off = sh * W + sw
                valid = (h + sh < H) & (w + sw < W)
                parts = []
                for j in (0, 1):
                    t = _shift_lanes(slabs[j + sd], off)
                    parts.append(jnp.where(valid, t, jnp.zeros_like(t)))
                rows.append(jnp.concatenate(parts, axis=1))
    xm = jnp.concatenate(rows, axis=0)
    y = jnp.dot(w_ref[...], xm, preferred_element_type=jnp.float32)
    yb = y.astype(jnp.bfloat16)
    y_ref[0] = yb[:, :H * W]
    y_ref[1] = yb[:, H * W:]
    sums = jnp.sum(y, axis=1, keepdims=True)
    sqs = jnp.sum(y * y, axis=1, keepdims=True)
    s_ref[...] = jnp.concatenate([sums, sqs], axis=1)[None]


def _phase_weight(w_up):
    sel = np.zeros((2, 2, 3), np.float32)
    sel[0, 0, 1] = 1.0
    sel[1, 0, 2] = 1.0
    sel[1, 1, 0] = 1.0
    sel = jnp.asarray(sel)
    w8 = jnp.einsum('PSa,QTb,RUc,ioabc->PQRoSTUi', sel, sel, sel,
                    w_up.astype(jnp.float32))
    Cout, Cin = w_up.shape[1], w_up.shape[0]
    return w8.reshape(8 * Cout, 8 * Cin)


def _res_kernel(hm_ref, c_ref, hp_ref, w_ref, z_ref, s_ref, *, TD, Do, Ho, Wo):
    do0 = (TD * pl.program_id(0)) % Do
    s_lo = jnp.where(do0 > 0, hm_ref[0], jnp.zeros_like(hm_ref[0]))
    s_hi = jnp.where(do0 < Do - TD, hp_ref[0], jnp.zeros_like(hp_ref[0]))
    slabs = (s_lo,) + tuple(c_ref[j] for j in range(TD)) + (s_hi,)

    lane = lax.broadcasted_iota(jnp.int32, (1, Ho * Wo), 1)
    h = lane // Wo
    w = lane % Wo
    rows = [None] * 27
    for kh in (0, 1, 2):
        for kw in (0, 1, 2):
            off = (kh - 1) * Wo + (kw - 1)
            valid = ((h + kh - 1 >= 0) & (h + kh - 1 < Ho)
                     & (w + kw - 1 >= 0) & (w + kw - 1 < Wo))
            shifted = [jnp.where(valid, _shift_lanes(s, off),
                                 jnp.zeros_like(s)) for s in slabs]
            for kd in (0, 1, 2):
                rows[(kd * 3 + kh) * 3 + kw] = jnp.concatenate(
                    shifted[kd:kd + TD], axis=1)
    xm = jnp.concatenate(rows, axis=0)
    z = jnp.dot(w_ref[...], xm, preferred_element_type=jnp.float32)
    zb = z.astype(jnp.bfloat16)
    for j in range(TD):
        z_ref[j] = zb[:, j * Ho * Wo:(j + 1) * Ho * Wo]
    sums = jnp.sum(z, axis=1, keepdims=True)
    sqs = jnp.sum(z * z, axis=1, keepdims=True)
    s_ref[...] = jnp.concatenate([sums, sqs], axis=1)[None]


def _final_kernel(z_ref, y_ref, sc2_ref, sh2_ref, al2_ref, o_ref):
    a2 = al2_ref[0, 0]
    for j in range(z_ref.shape[0]):
        t2 = (z_ref[j].astype(jnp.float32) * sc2_ref[...] + sh2_ref[...])
        o_ref[j] = (jnp.where(t2 > 0, t2, a2 * t2)
                    + y_ref[j].astype(jnp.float32))


def _finalize_bn(s, count, gamma, beta):
    mean = s[:, 0] / count
    var = jnp.maximum(s[:, 1] / count - mean * mean, 0.0)
    scale = gamma.astype(jnp.float32) * lax.rsqrt(var + _EPS)
    shift = beta.astype(jnp.float32) - mean * scale
    return scale.reshape(-1, 1), shift.reshape(-1, 1)


def kernel(x, w_up, b_up, gamma1, beta1, alpha1,
           w_res, b_res, gamma2, beta2, alpha2):
    N, Cin, D, H, W = x.shape
    Cout = w_up.shape[1]
    Do, Ho, Wo = 2 * D, 2 * H, 2 * W
    HW, HWo = H * W, Ho * Wo
    count = N * Do * HWo

    x_t = (x.astype(jnp.bfloat16)
           .transpose(0, 2, 1, 3, 4).reshape(N * D, Cin, HW))
    w8 = _phase_weight(w_up).astype(jnp.bfloat16)
    nd = N * D
    ga = nd // 2
    y_ph, st1 = pl.pallas_call(
        functools.partial(_up_kernel, D=D, H=H, W=W),
        out_shape=(jax.ShapeDtypeStruct((nd, 8 * Cout, HW), jnp.bfloat16),
                   jax.ShapeDtypeStruct((ga, 8 * Cout, 2), jnp.float32)),
        grid=(ga,),
        in_specs=[
            pl.BlockSpec((2, Cin, HW), lambda i: (i, 0, 0)),
            pl.BlockSpec((1, Cin, HW),
                         lambda i: (jnp.minimum(2 * i + 2, nd - 1), 0, 0)),
            pl.BlockSpec((8 * Cout, 8 * Cin), lambda i: (0, 0)),
        ],
        out_specs=(pl.BlockSpec((2, 8 * Cout, HW), lambda i: (i, 0, 0)),
                   pl.BlockSpec((1, 8 * Cout, 2), lambda i: (i, 0, 0))),
        compiler_params=pltpu.CompilerParams(dimension_semantics=("parallel",)),
    )(x_t, x_t, w8)

    s1 = st1.sum(axis=0).reshape(8, Cout, 2).sum(axis=0)
    sc1, sh1 = _finalize_bn(s1, count, gamma1, beta1)

    y_d = (y_ph.reshape(N, D, 2, 2, 2, Cout, H, W)
           .transpose(0, 1, 2, 5, 6, 3, 7, 4)
           .reshape(N * Do, Cout, HWo))
    t1 = y_d.astype(jnp.float32) * sc1[None] + sh1[None]
    y_act = jnp.where(t1 > 0, t1, alpha1 * t1).astype(jnp.bfloat16)

    w_r = jnp.transpose(w_res, (0, 2, 3, 4, 1)).reshape(Cout, 27 * Cout)
    w_r = w_r.astype(jnp.bfloat16)
    nrow = N * Do
    TD = 4
    gb = nrow // TD
    z, st2 = pl.pallas_call(
        functools.partial(_res_kernel, TD=TD, Do=Do, Ho=Ho, Wo=Wo),
        out_shape=(jax.ShapeDtypeStruct((nrow, Cout, HWo), jnp.bfloat16),
                   jax.ShapeDtypeStruct((gb, Cout, 2), jnp.float32)),
        grid=(gb,),
        in_specs=[
            pl.BlockSpec((1, Cout, HWo),
                         lambda i: (jnp.maximum(TD * i - 1, 0), 0, 0)),
            pl.BlockSpec((TD, Cout, HWo), lambda i: (i, 0, 0)),
            pl.BlockSpec((1, Cout, HWo),
                         lambda i: (jnp.minimum(TD * i + TD, nrow - 1), 0, 0)),
            pl.BlockSpec((Cout, 27 * Cout), lambda i: (0, 0)),
        ],
        out_specs=(pl.BlockSpec((TD, Cout, HWo), lambda i: (i, 0, 0)),
                   pl.BlockSpec((1, Cout, 2), lambda i: (i, 0, 0))),
        compiler_params=pltpu.CompilerParams(dimension_semantics=("parallel",)),
    )(y_act, y_act, y_act, w_r)

    sc2, sh2 = _finalize_bn(st2.sum(axis=0), count, gamma2, beta2)

    out_s = pl.pallas_call(
        _final_kernel,
        out_shape=jax.ShapeDtypeStruct((nrow, Cout, HWo), jnp.float32),
        grid=(gb,),
        in_specs=[
            pl.BlockSpec((TD, Cout, HWo), lambda i: (i, 0, 0)),
            pl.BlockSpec((TD, Cout, HWo), lambda i: (i, 0, 0)),
            pl.BlockSpec((Cout, 1), lambda i: (0, 0)),
            pl.BlockSpec((Cout, 1), lambda i: (0, 0)),
            pl.BlockSpec((1, 1), lambda i: (0, 0)),
        ],
        out_specs=pl.BlockSpec((TD, Cout, HWo), lambda i: (i, 0, 0)),
        compiler_params=pltpu.CompilerParams(dimension_semantics=("parallel",)),
    )(z, y_act, sc2, sh2, jnp.full((1, 1), alpha2, jnp.float32))

    return (out_s.reshape(N, Do, Cout, HWo).transpose(0, 2, 1, 3)
            .reshape(N, Cout, Do, Ho, Wo))
